# baseline (device time: 13927 ns/iter reference)
import jax
import jax.numpy as jnp
from jax import lax
from jax.experimental import pallas as pl
from jax.experimental.pallas import tpu as pltpu

N_DEV = 8
E_PER = 2


def kernel(x, router_W, route_idx, expert_W):
    del router_W
    n, d = x.shape
    h = expert_W.shape[-1]
    rows = n // N_DEV

    def body(x_ref, idx_ref, w_ref, out_ref,
             part_buf, rs_comm, red_buf, ag_comm,
             rs_send_sems, rs_recv_sems, ag_send_sems, ag_recv_sems):
        me = lax.axis_index("i")

        barrier_sem = pltpu.get_barrier_semaphore()
        for p in range(N_DEV):
            @pl.when(me != p)
            def _(p=p):
                pl.semaphore_signal(
                    barrier_sem, inc=1,
                    device_id=(p,), device_id_type=pl.DeviceIdType.MESH,
                )
        pl.semaphore_wait(barrier_sem, N_DEV - 1)

        e0 = me * E_PER
        acc = jnp.zeros((n, h), jnp.float32)
        for k in range(E_PER):
            xm = jnp.where(idx_ref[:, :] == e0 + k, x_ref[:, :], 0.0)
            acc = acc + jnp.dot(
                xm.astype(jnp.bfloat16),
                w_ref[k, :, :].astype(jnp.bfloat16),
                preferred_element_type=jnp.float32,
            )
        part_buf[:, :] = acc.astype(jnp.bfloat16)

        def rs_rdma(p):
            return pltpu.make_async_remote_copy(
                src_ref=part_buf.at[pl.ds(p * rows, rows), :],
                dst_ref=rs_comm.at[me],
                send_sem=rs_send_sems.at[p],
                recv_sem=rs_recv_sems.at[me],
                device_id=(p,),
                device_id_type=pl.DeviceIdType.MESH,
            )

        for p in range(N_DEV):
            @pl.when(me != p)
            def _(p=p):
                rs_rdma(p).start()

        red_buf[:, :] = part_buf[pl.ds(me * rows, rows), :]

        for s in range(N_DEV):
            @pl.when(me != s)
            def _(s=s):
                recv = pltpu.make_async_remote_copy(
                    src_ref=part_buf.at[pl.ds(s * rows, rows), :],
                    dst_ref=rs_comm.at[s],
                    send_sem=rs_send_sems.at[s],
                    recv_sem=rs_recv_sems.at[s],
                    device_id=(s,),
                    device_id_type=pl.DeviceIdType.MESH,
                )
                recv.wait_recv()
                red_buf[:, :] += rs_comm[s, :, :]

        out_ref[pl.ds(me * rows, rows), :] = red_buf[:, :].astype(jnp.float32)

        for p in range(N_DEV):
            @pl.when(me != p)
            def _(p=p):
                rs_rdma(p).wait_send()

    return pl.pallas_call(
        body,
        out_shape=jax.ShapeDtypeStruct((n, h), jnp.float32),
        in_specs=[pl.BlockSpec(memory_space=pltpu.VMEM)] * 3,
        out_specs=pl.BlockSpec(memory_space=pltpu.VMEM),
        scratch_shapes=[
            pltpu.VMEM((n, h), jnp.bfloat16),
            pltpu.VMEM((N_DEV, rows, h), jnp.bfloat16),
            pltpu.VMEM((rows, h), jnp.bfloat16),
            pltpu.VMEM((N_DEV, rows, h), jnp.bfloat16),
            pltpu.SemaphoreType.DMA((N_DEV,)),
            pltpu.SemaphoreType.DMA((N_DEV,)),
            pltpu.SemaphoreType.DMA((N_DEV,)),
            pltpu.SemaphoreType.DMA((N_DEV,)),
        ],
        compiler_params=pltpu.CompilerParams(collective_id=0),
    )(x, route_idx, expert_W)
